# Initial kernel scaffold; baseline (speedup 1.0000x reference)
import jax
import jax.numpy as jnp
from jax.experimental import pallas as pl


def kernel(z, in_v, in_g, in_b, out_v, out_g, out_b, codebooks):
    raise NotImplementedError("write your pallas kernel here")


# speedup vs baseline: 4.8356x; 4.8356x over previous
import jax
import jax.numpy as jnp
from jax.experimental import pallas as pl
from jax.experimental.pallas import tpu as pltpu

_NC = 9
_CS = 1024
_CD = 8
_D = 512
_TB = 1024


def _dot_bf16(a, b):
    return jax.lax.dot_general(
        a.astype(jnp.bfloat16), b.astype(jnp.bfloat16),
        (((1,), (0,)), ((), ())), preferred_element_type=jnp.float32)


def _dot_f32(a, b):
    return jax.lax.dot_general(a, b, (((1,), (0,)), ((), ())),
                               preferred_element_type=jnp.float32)


def _rvq_kernel(z_ref, win_ref, wout_ref, cbn_ref, cbt_ref, ncb_ref,
                inb_ref, outb_ref, zqt_ref, codes_ref, lat_ref, loss_ref,
                res_scr):
    res_scr[...] = z_ref[0]
    zqt_ref[0] = jnp.zeros_like(zqt_ref[0])
    loss = jnp.float32(0.0)
    for i in range(_NC):
        r0 = _CD * i
        wi = win_ref[r0:r0 + _CD, :]
        ze = _dot_bf16(wi, res_scr[...]) + inb_ref[r0:r0 + _CD, :]
        lat_ref[0, r0:r0 + _CD, :] = ze
        nsq = jnp.sum(ze * ze, axis=0, keepdims=True)
        enc_n = ze / jnp.maximum(jnp.sqrt(nsq), 1e-12)
        nself = jnp.sum(enc_n * enc_n, axis=0, keepdims=True)
        cross = _dot_bf16(cbn_ref[i], enc_n)
        dist = (nself - 2.0 * cross) + ncb_ref[_CS * i:_CS * (i + 1), :]
        idx = jnp.argmin(dist, axis=0)
        codes_ref[0, i, :] = idx
        iota = jax.lax.broadcasted_iota(jnp.int32, dist.shape, 0)
        onehot = (iota == idx[None, :]).astype(jnp.float32)
        zq = _dot_f32(cbt_ref[i], onehot)
        d = ze - zq
        loss = loss + jnp.sum(d * d)
        zq_st = ze + (zq - ze)
        wo = wout_ref[:, r0:r0 + _CD]
        out = _dot_bf16(wo, zq_st) + outb_ref[:, i:i + 1]
        zqt_ref[0] = zqt_ref[0] + out
        res_scr[...] = res_scr[...] - out
    loss_ref[0, 0] = jnp.full((8, 128), loss * (1.0 / 1024.0), jnp.float32)


def kernel(z, in_v, in_g, in_b, out_v, out_g, out_b, codebooks):
    B, D, T = z.shape
    nc, cs, cd = codebooks.shape
    f32 = jnp.float32

    n_in = jnp.sqrt(jnp.sum(in_v * in_v, axis=2, keepdims=True))
    win = (in_g[..., None] * in_v / n_in).reshape(nc * cd, D)
    n_out = jnp.sqrt(jnp.sum(out_v * out_v, axis=2, keepdims=True))
    wo3 = out_g[..., None] * out_v / n_out
    wout = jnp.transpose(wo3, (1, 0, 2)).reshape(D, nc * cd)
    cb_norm = jnp.sqrt(jnp.sum(codebooks * codebooks, axis=2, keepdims=True))
    cbn = codebooks / jnp.maximum(cb_norm, 1e-12)
    ncb = jnp.sum(cbn * cbn, axis=2).reshape(nc * cs, 1)
    cbt = jnp.transpose(codebooks, (0, 2, 1))
    inb = in_b.reshape(nc * cd, 1)
    outb = out_b.T

    tb = _TB if T % _TB == 0 else T
    ntb = T // tb
    grid = (B, ntb)
    out_shapes = (
        jax.ShapeDtypeStruct((B, D, T), f32),
        jax.ShapeDtypeStruct((B, nc, T), jnp.int32),
        jax.ShapeDtypeStruct((B, nc * cd, T), f32),
        jax.ShapeDtypeStruct((B, ntb, 8, 128), f32),
    )
    zqt, codes, latents, lpart = pl.pallas_call(
        _rvq_kernel,
        grid=grid,
        in_specs=[
            pl.BlockSpec((1, D, tb), lambda b, t: (b, 0, t)),
            pl.BlockSpec((nc * cd, D), lambda b, t: (0, 0)),
            pl.BlockSpec((D, nc * cd), lambda b, t: (0, 0)),
            pl.BlockSpec((nc, cs, cd), lambda b, t: (0, 0, 0)),
            pl.BlockSpec((nc, cd, cs), lambda b, t: (0, 0, 0)),
            pl.BlockSpec((nc * cs, 1), lambda b, t: (0, 0)),
            pl.BlockSpec((nc * cd, 1), lambda b, t: (0, 0)),
            pl.BlockSpec((D, nc), lambda b, t: (0, 0)),
        ],
        out_specs=[
            pl.BlockSpec((1, D, tb), lambda b, t: (b, 0, t)),
            pl.BlockSpec((1, nc, tb), lambda b, t: (b, 0, t)),
            pl.BlockSpec((1, nc * cd, tb), lambda b, t: (b, 0, t)),
            pl.BlockSpec((1, 1, 8, 128), lambda b, t: (b, t, 0, 0)),
        ],
        out_shape=out_shapes,
        scratch_shapes=[pltpu.VMEM((D, tb), f32)],
    )(z, win, wout, cbn, cbt, ncb, inb, outb)

    loss = jnp.sum(lpart) * (1.0 / (B * cd * T))
    return (zqt, codes, latents, loss, loss)
